# baseline (device time: 12171 ns/iter reference)
import jax
import jax.numpy as jnp
from jax import lax
from jax.experimental import pallas as pl
from jax.experimental.pallas import tpu as pltpu

N_DEV = 4


def kernel(x, dy, gamma):
    m, d = x.shape

    def body(x_hbm, dy_hbm, out_hbm, xv_ref, dyv_ref, comm_ref,
             copy_sems, send_sems, recv_sems):
        my = lax.axis_index("i")
        barrier_sem = pltpu.get_barrier_semaphore()

        cp_x = pltpu.make_async_copy(x_hbm, xv_ref, copy_sems.at[0])
        cp_dy = pltpu.make_async_copy(dy_hbm, dyv_ref, copy_sems.at[1])
        cp_x.start()
        cp_dy.start()

        for k in range(1, N_DEV):
            pl.semaphore_signal(
                barrier_sem, inc=1,
                device_id=((my + k) % N_DEV,),
                device_id_type=pl.DeviceIdType.MESH,
            )

        cp_x.wait()
        xv = xv_ref[:, :].astype(jnp.bfloat16)
        sx = jnp.sum(xv, axis=1, keepdims=True, dtype=jnp.float32)
        sxx = jnp.sum(xv * xv, axis=1, keepdims=True, dtype=jnp.float32)
        mu = sx * (1.0 / d)
        var = sxx * (1.0 / d) - mu * mu
        rstd = lax.rsqrt(var + 1e-5)
        xhat = (xv - mu.astype(jnp.bfloat16)) * rstd.astype(jnp.bfloat16)

        cp_dy.wait()
        dyv = dyv_ref[:, :].astype(jnp.bfloat16)
        dgamma = jnp.sum(dyv * xhat, axis=0, dtype=jnp.float32)
        dbeta = jnp.sum(dyv, axis=0, dtype=jnp.float32)
        comm_ref[0, :, :] = jnp.stack([dgamma, dbeta])

        pl.semaphore_wait(barrier_sem, N_DEV - 1)

        sends = []
        for k in range(1, N_DEV):
            rdma = pltpu.make_async_remote_copy(
                src_ref=comm_ref.at[0],
                dst_ref=comm_ref.at[k],
                send_sem=send_sems.at[k - 1],
                recv_sem=recv_sems.at[k - 1],
                device_id=((my + k) % N_DEV,),
                device_id_type=pl.DeviceIdType.MESH,
            )
            rdma.start()
            sends.append(rdma)
        for rdma in sends:
            rdma.wait_recv()

        comm_ref[0, :, :] = (
            (comm_ref[0] + comm_ref[1]) + (comm_ref[2] + comm_ref[3])
        )
        cp_out = pltpu.make_async_copy(
            comm_ref.at[0], out_hbm, copy_sems.at[2]
        )
        cp_out.start()
        for rdma in sends:
            rdma.wait_send()
        cp_out.wait()

    return pl.pallas_call(
        body,
        out_shape=jax.ShapeDtypeStruct((2, d), jnp.float32),
        in_specs=[
            pl.BlockSpec(memory_space=pltpu.HBM),
            pl.BlockSpec(memory_space=pltpu.HBM),
        ],
        out_specs=pl.BlockSpec(memory_space=pltpu.HBM),
        scratch_shapes=[
            pltpu.VMEM((m, d), jnp.float32),
            pltpu.VMEM((m, d), jnp.float32),
            pltpu.VMEM((N_DEV, 2, d), jnp.float32),
            pltpu.SemaphoreType.DMA((3,)),
            pltpu.SemaphoreType.DMA((N_DEV - 1,)),
            pltpu.SemaphoreType.DMA((N_DEV - 1,)),
        ],
        compiler_params=pltpu.CompilerParams(collective_id=0),
    )(x, dy)


# device time: 9148 ns/iter; 1.3305x vs baseline; 1.3305x over previous
import jax
import jax.numpy as jnp
from jax import lax
from jax.experimental import pallas as pl
from jax.experimental.pallas import tpu as pltpu

N_DEV = 4


def kernel(x, dy, gamma):
    m, d = x.shape
    x = pltpu.with_memory_space_constraint(x, pltpu.HBM)
    dy = pltpu.with_memory_space_constraint(dy, pltpu.HBM)

    C = 4
    bm = m // C

    def body(x_hbm, dy_hbm, out_ref, xv_ref, dyv_ref, comm_ref,
             copy_sems, send_sems, recv_sems):
        my = lax.axis_index("i")
        barrier_sem = pltpu.get_barrier_semaphore()

        cps = []
        for c in range(C):
            rows = pl.ds(c * bm, bm)
            cp_x = pltpu.make_async_copy(
                x_hbm.at[rows], xv_ref.at[rows], copy_sems.at[2 * c]
            )
            cp_dy = pltpu.make_async_copy(
                dy_hbm.at[rows], dyv_ref.at[rows], copy_sems.at[2 * c + 1]
            )
            cp_x.start()
            cp_dy.start()
            cps.append((cp_x, cp_dy))

        for k in range(1, N_DEV):
            pl.semaphore_signal(
                barrier_sem, inc=1,
                device_id=((my + k) % N_DEV,),
                device_id_type=pl.DeviceIdType.MESH,
            )

        dgamma = jnp.zeros((d,), jnp.float32)
        dbeta = jnp.zeros((d,), jnp.float32)
        for c in range(C):
            rows = pl.ds(c * bm, bm)
            cps[c][0].wait()
            xv = xv_ref[rows, :].astype(jnp.bfloat16)
            sx = jnp.sum(xv, axis=1, keepdims=True, dtype=jnp.float32)
            sxx = jnp.sum(xv * xv, axis=1, keepdims=True, dtype=jnp.float32)
            mu = sx * (1.0 / d)
            var = sxx * (1.0 / d) - mu * mu
            rstd = lax.rsqrt(var + 1e-5)
            xhat = (xv - mu.astype(jnp.bfloat16)) * rstd.astype(jnp.bfloat16)
            cps[c][1].wait()
            dyv = dyv_ref[rows, :].astype(jnp.bfloat16)
            dgamma += jnp.sum(dyv * xhat, axis=0, dtype=jnp.float32)
            dbeta += jnp.sum(dyv, axis=0, dtype=jnp.float32)
        comm_ref[0, 0, :] = dgamma
        comm_ref[0, 1, :] = dbeta

        pl.semaphore_wait(barrier_sem, N_DEV - 1)

        sends = []
        for k in range(1, N_DEV):
            rdma = pltpu.make_async_remote_copy(
                src_ref=comm_ref.at[0],
                dst_ref=comm_ref.at[k],
                send_sem=send_sems.at[k - 1],
                recv_sem=recv_sems.at[k - 1],
                device_id=((my + k) % N_DEV,),
                device_id_type=pl.DeviceIdType.MESH,
            )
            rdma.start()
            sends.append(rdma)

        sends[0].wait_recv()
        sends[2].wait_recv()
        acc = (comm_ref[0] + comm_ref[1]) + comm_ref[3]
        sends[1].wait_recv()
        out_ref[:, :] = acc + comm_ref[2]
        for rdma in sends:
            rdma.wait_send()

    return pl.pallas_call(
        body,
        out_shape=jax.ShapeDtypeStruct((2, d), jnp.float32),
        in_specs=[
            pl.BlockSpec(memory_space=pltpu.HBM),
            pl.BlockSpec(memory_space=pltpu.HBM),
        ],
        out_specs=pl.BlockSpec(memory_space=pltpu.VMEM),
        scratch_shapes=[
            pltpu.VMEM((m, d), jnp.float32),
            pltpu.VMEM((m, d), jnp.float32),
            pltpu.VMEM((N_DEV, 2, d), jnp.float32),
            pltpu.SemaphoreType.DMA((2 * 4,)),
            pltpu.SemaphoreType.DMA((N_DEV - 1,)),
            pltpu.SemaphoreType.DMA((N_DEV - 1,)),
        ],
        compiler_params=pltpu.CompilerParams(collective_id=0),
    )(x, dy)
